# baseline (speedup 1.0000x reference)
import functools

import jax
import jax.numpy as jnp
from jax import lax
from jax.experimental import pallas as pl
from jax.experimental.pallas import tpu as pltpu
from jax.experimental.pallas import tpu_sc as plsc

_INFO = plsc.get_sparse_core_info()
_NC = _INFO.num_cores
_NS = _INFO.num_subcores
_NW = _NC * _NS
_NBUF = 6
_LANES = 16
_BLKCOLS = 128


@functools.lru_cache(maxsize=None)
def _make_gather(vocab, dim, batch):
    b_per_w = batch // _NW
    blk = b_per_w * dim
    pad = b_per_w + _BLKCOLS + 2 * _LANES
    mesh = plsc.VectorSubcoreMesh(core_axis_name="c", subcore_axis_name="s")
    iota = lambda: lax.iota(jnp.int32, _LANES)

    @functools.partial(
        pl.kernel,
        mesh=mesh,
        compiler_params=pltpu.CompilerParams(needs_layout_passes=False),
        out_type=jax.ShapeDtypeStruct((batch * dim,), jnp.float32),
        scratch_types=[
            pltpu.VMEM((pad,), jnp.int32),
            pltpu.VMEM((pad,), jnp.int32),
            pltpu.VMEM((_NBUF, dim, _BLKCOLS), jnp.float32),
            pltpu.VMEM((blk,), jnp.float32),
            [pltpu.SemaphoreType.DMA] * _NBUF,
        ],
    )
    def gather(table_t_hbm, sidx, out_hbm, sidx_v, fpos_v, blks_v, rows_v, sems):
        wid = lax.axis_index("s") * _NC + lax.axis_index("c")
        pltpu.sync_copy(sidx.at[wid], sidx_v.at[pl.ds(_BLKCOLS, b_per_w)])
        sidx_v[pl.ds(_BLKCOLS - _LANES, _LANES)] = jnp.broadcast_to(-1, (_LANES,))

        def sorted_at(i):
            return sidx_v[pl.ds(i + _BLKCOLS, _LANES)][0]

        def scan_body(g, k):
            cur = sidx_v[pl.ds(g * _LANES + _BLKCOLS, _LANES)]
            prev = sidx_v[pl.ds(g * _LANES + _BLKCOLS - 1, _LANES)]
            chg = (cur >> 7) != (prev >> 7)
            plsc.store_compressed(
                fpos_v.at[pl.ds(k, _LANES)], iota() + g * _LANES, mask=chg
            )
            return k + plsc.all_reduce_population_count(chg)[0]

        nf = lax.fori_loop(0, b_per_w // _LANES, scan_body, 0)
        plsc.store_scatter(
            fpos_v,
            [jnp.broadcast_to(nf, (_LANES,))],
            jnp.broadcast_to(b_per_w, (_LANES,)),
            mask=iota() == 0,
        )

        def fire(f, j):
            @pl.when(f < nf)
            def _():
                pos = fpos_v[pl.ds(f, _LANES)][0]
                base = pl.multiple_of((sorted_at(pos) >> 7) << 7, _BLKCOLS)
                pltpu.async_copy(
                    table_t_hbm.at[:, pl.ds(base, _BLKCOLS)],
                    blks_v.at[j],
                    sems[j],
                )

        def extract_fetch(f, j):
            pos = fpos_v[pl.ds(f, _LANES)][0]
            nxt = fpos_v[pl.ds(f + 1, _LANES)][0]
            tcb = (sorted_at(pos) >> 7) << 7

            def inner(e, carry):
                i = pos + e
                lane = jnp.broadcast_to(sorted_at(i) - tcb, (_LANES,))
                for u in range(dim // _LANES):
                    rows = iota() + (u * _LANES)
                    vals = plsc.load_gather(blks_v.at[j], [rows, lane])
                    rows_v[pl.ds(i * dim + u * _LANES, _LANES)] = vals
                return carry

            lax.fori_loop(0, nxt - pos, inner, 0)

        for j in range(_NBUF):
            fire(j, j)

        def round_body(r, carry):
            f0 = r * _NBUF
            for j in range(_NBUF):
                f = f0 + j

                @pl.when(f < nf)
                def _():
                    pltpu.make_async_copy(
                        table_t_hbm.at[:, pl.ds(0, _BLKCOLS)],
                        blks_v.at[j],
                        sems[j],
                    ).wait()
                    extract_fetch(f, j)

                fire(f + _NBUF, j)
            return carry

        lax.fori_loop(0, (nf + _NBUF - 1) // _NBUF, round_body, 0)
        pltpu.sync_copy(rows_v, out_hbm.at[pl.ds(wid * blk, blk)])

    return gather


def kernel(class_label, table):
    batch = class_label.shape[0]
    vocab, dim = table.shape
    idx32 = class_label.astype(jnp.int32)
    sorted_idx, order = lax.sort(
        (idx32, jnp.arange(batch, dtype=jnp.int32)), num_keys=1
    )
    out = _make_gather(vocab, dim, batch)(
        table.T, sorted_idx.reshape(_NW, batch // _NW)
    )
    rows = out.reshape(batch, dim)
    _, inv = lax.sort(
        (order, jnp.arange(batch, dtype=jnp.int32)), num_keys=1
    )
    return rows[inv].reshape(batch, 1, dim)


# speedup vs baseline: 1.0158x; 1.0158x over previous
import functools

import jax
import jax.numpy as jnp
from jax import lax
from jax.experimental import pallas as pl
from jax.experimental.pallas import tpu as pltpu
from jax.experimental.pallas import tpu_sc as plsc

_INFO = plsc.get_sparse_core_info()
_NC = _INFO.num_cores
_NS = _INFO.num_subcores
_NW = _NC * _NS
_NBUF = 8
_LANES = 16
_BLKCOLS = 128


@functools.lru_cache(maxsize=None)
def _make_gather(vocab, dim, batch):
    b_per_w = batch // _NW
    blk = b_per_w * dim
    pad = b_per_w + _BLKCOLS + 2 * _LANES
    mesh = plsc.VectorSubcoreMesh(core_axis_name="c", subcore_axis_name="s")
    iota = lambda: lax.iota(jnp.int32, _LANES)

    @functools.partial(
        pl.kernel,
        mesh=mesh,
        compiler_params=pltpu.CompilerParams(needs_layout_passes=False),
        out_type=jax.ShapeDtypeStruct((batch * dim,), jnp.float32),
        scratch_types=[
            pltpu.VMEM((pad,), jnp.int32),
            pltpu.VMEM((pad,), jnp.int32),
            pltpu.VMEM((_NBUF, dim, _BLKCOLS), jnp.float32),
            pltpu.VMEM((blk,), jnp.float32),
            [pltpu.SemaphoreType.DMA] * _NBUF,
        ],
    )
    def gather(table_t_hbm, sidx, out_hbm, sidx_v, fpos_v, blks_v, rows_v, sems):
        wid = lax.axis_index("s") * _NC + lax.axis_index("c")
        pltpu.sync_copy(sidx.at[wid], sidx_v.at[pl.ds(_BLKCOLS, b_per_w)])
        sidx_v[pl.ds(_BLKCOLS - _LANES, _LANES)] = jnp.broadcast_to(-1, (_LANES,))

        def sorted_at(i):
            return sidx_v[pl.ds(i + _BLKCOLS, _LANES)][0]

        def scan_body(g, k):
            cur = sidx_v[pl.ds(g * _LANES + _BLKCOLS, _LANES)]
            prev = sidx_v[pl.ds(g * _LANES + _BLKCOLS - 1, _LANES)]
            chg = (cur >> 7) != (prev >> 7)
            plsc.store_compressed(
                fpos_v.at[pl.ds(k, _LANES)], iota() + g * _LANES, mask=chg
            )
            return k + plsc.all_reduce_population_count(chg)[0]

        nf = lax.fori_loop(0, b_per_w // _LANES, scan_body, 0)
        plsc.store_scatter(
            fpos_v,
            [jnp.broadcast_to(nf, (_LANES,))],
            jnp.broadcast_to(b_per_w, (_LANES,)),
            mask=iota() == 0,
        )

        def fire(f, j):
            @pl.when(f < nf)
            def _():
                pos = fpos_v[pl.ds(f, _LANES)][0]
                base = pl.multiple_of((sorted_at(pos) >> 7) << 7, _BLKCOLS)
                pltpu.async_copy(
                    table_t_hbm.at[:, pl.ds(base, _BLKCOLS)],
                    blks_v.at[j],
                    sems[j],
                )

        def extract_fetch(f, j):
            pos = fpos_v[pl.ds(f, _LANES)][0]
            nxt = fpos_v[pl.ds(f + 1, _LANES)][0]
            tcb = (sorted_at(pos) >> 7) << 7

            def inner(e, carry):
                i = pos + e
                lane = jnp.broadcast_to(sorted_at(i) - tcb, (_LANES,))
                for u in range(dim // _LANES):
                    rows = iota() + (u * _LANES)
                    vals = plsc.load_gather(blks_v.at[j], [rows, lane])
                    rows_v[pl.ds(i * dim + u * _LANES, _LANES)] = vals
                return carry

            lax.fori_loop(0, nxt - pos, inner, 0)

        for j in range(_NBUF):
            fire(j, j)

        def round_body(r, carry):
            f0 = r * _NBUF
            for j in range(_NBUF):
                f = f0 + j

                @pl.when(f < nf)
                def _():
                    pltpu.make_async_copy(
                        table_t_hbm.at[:, pl.ds(0, _BLKCOLS)],
                        blks_v.at[j],
                        sems[j],
                    ).wait()
                    extract_fetch(f, j)

                fire(f + _NBUF, j)
            return carry

        lax.fori_loop(0, (nf + _NBUF - 1) // _NBUF, round_body, 0)
        pltpu.sync_copy(rows_v, out_hbm.at[pl.ds(wid * blk, blk)])

    return gather


def kernel(class_label, table):
    batch = class_label.shape[0]
    vocab, dim = table.shape
    idx32 = class_label.astype(jnp.int32)
    sorted_idx, order = lax.sort(
        (idx32, jnp.arange(batch, dtype=jnp.int32)), num_keys=1
    )
    out = _make_gather(vocab, dim, batch)(
        table.T, sorted_idx.reshape(_NW, batch // _NW)
    )
    rows = out.reshape(batch, dim)
    _, inv = lax.sort(
        (order, jnp.arange(batch, dtype=jnp.int32)), num_keys=1
    )
    return rows[inv].reshape(batch, 1, dim)
